# baseline (speedup 1.0000x reference)
import jax
import jax.numpy as jnp
from jax.experimental import pallas as pl

B_BLK = 16


def _body(x_ref, pe_ref, o_ref):
    o_ref[...] = x_ref[...] + pe_ref[...][None, :, :]


def kernel(x, pos_embed):
    B, L, D = x.shape
    grid = (B // B_BLK,)
    return pl.pallas_call(
        _body,
        grid=grid,
        in_specs=[
            pl.BlockSpec((B_BLK, L, D), lambda i: (i, 0, 0)),
            pl.BlockSpec((L, D), lambda i: (0, 0)),
        ],
        out_specs=pl.BlockSpec((B_BLK, L, D), lambda i: (i, 0, 0)),
        out_shape=jax.ShapeDtypeStruct((B, L, D), x.dtype),
    )(x, pos_embed)
